# Initial kernel scaffold; baseline (speedup 1.0000x reference)
import jax
import jax.numpy as jnp
from jax.experimental import pallas as pl


def kernel(X, positive_edges, negative_edges, labels, label_mask, Wpb, bpb, Wnb, bnb, Wpd, bpd, Wnd, bnd):
    raise NotImplementedError("write your pallas kernel here")


# speedup vs baseline: 8.1005x; 8.1005x over previous
import functools

import jax
import jax.numpy as jnp
from jax import lax
from jax.experimental import pallas as pl
from jax.experimental.pallas import tpu as pltpu
from jax.experimental.pallas import tpu_sc as plsc

N = 4096
D = 2048
E = 65536
H = 64

F32 = jnp.float32

NC, NS = 2, 16
NW = NC * NS
CH = 128
EPW = E // NW
NCHUNK = EPW // CH
TRASH = N
NR = N + 128
ZR = NR // NS
WR = N // NS
FB = 128
FD = 128

BM = 256


def _normalize_rows(x):
    n = jnp.sqrt(jnp.sum(x * x, axis=1, keepdims=True))
    return x / jnp.maximum(n, 1e-12)


def _project_body(x_ref, w_ref, gp_ref, gn_ref, ys_ref):
    p = jnp.dot(x_ref[...], w_ref[...], preferred_element_type=F32)
    ones = jnp.ones((BM, FB - H), F32)
    gp_ref[...] = jnp.concatenate([p[:, :H], ones], axis=1)
    gn_ref[...] = jnp.concatenate([p[:, H:2 * H], ones], axis=1)
    ys_ref[...] = p[:, 2 * H:]


def _project(X, Wcat):
    grid = (N // BM,)
    return pl.pallas_call(
        _project_body,
        grid=grid,
        in_specs=[
            pl.BlockSpec((BM, D), lambda i: (i, 0)),
            pl.BlockSpec((D, 4 * H), lambda i: (0, 0)),
        ],
        out_specs=[
            pl.BlockSpec((BM, FB), lambda i: (i, 0)),
            pl.BlockSpec((BM, FB), lambda i: (i, 0)),
            pl.BlockSpec((BM, 2 * H), lambda i: (i, 0)),
        ],
        out_shape=[
            jax.ShapeDtypeStruct((N, FB), F32),
            jax.ShapeDtypeStruct((N, FB), F32),
            jax.ShapeDtypeStruct((N, 2 * H), F32),
        ],
    )(X, Wcat)


def _sc_base_body(gp, gn, ep, en, z80, accp_out, accn_out, radj_out,
                  idxr, idxc, radj, gbuf, accp, accn, sem):
    c = lax.axis_index("c")
    s = lax.axis_index("s")
    wid = s * NC + c
    pltpu.sync_copy(z80, accp.at[pl.ds(s * ZR, ZR)])
    pltpu.sync_copy(z80, accn.at[pl.ds(s * ZR, ZR)])
    plsc.subcore_barrier()

    for eref, gref, acc, setid in ((ep, gp, accp, 0), (en, gn, accn, 1)):
        ebase = wid * EPW

        def body(k, carry, eref=eref, gref=gref, acc=acc, setid=setid,
                 ebase=ebase):
            base = ebase + k * CH
            pltpu.sync_copy(eref.at[0, pl.ds(base, CH)], idxr)
            pltpu.sync_copy(eref.at[1, pl.ds(base, CH)], idxc)

            def adj(j, carry2):
                r = idxr[pl.ds(j * 16, 16)]
                cc = idxc[pl.ds(j * 16, 16)]
                radj[pl.ds(j * 16, 16)] = jnp.where(r == cc, TRASH, r)
                return carry2

            lax.fori_loop(0, CH // 16, adj, 0)
            pltpu.async_copy(gref.at[idxc], gbuf, sem).wait()
            pltpu.sync_copy(gbuf, acc.at[radj], add=True)
            pltpu.sync_copy(radj, radj_out.at[setid, pl.ds(base, CH)])
            return carry

        lax.fori_loop(0, NCHUNK, body, 0)

    plsc.subcore_barrier()
    pltpu.sync_copy(accp.at[pl.ds(s * WR, WR)],
                    accp_out.at[c, pl.ds(s * WR, WR)])
    pltpu.sync_copy(accn.at[pl.ds(s * WR, WR)],
                    accn_out.at[c, pl.ds(s * WR, WR)])


def _sc_base(gp, gn, ep, en):
    z80 = jnp.zeros((ZR, FB), F32)
    mesh = plsc.VectorSubcoreMesh(core_axis_name="c", subcore_axis_name="s")
    fn = functools.partial(
        pl.kernel,
        mesh=mesh,
        out_type=[
            jax.ShapeDtypeStruct((NC, N, FB), F32),
            jax.ShapeDtypeStruct((NC, N, FB), F32),
            jax.ShapeDtypeStruct((2, E), jnp.int32),
        ],
        scratch_types=[
            pltpu.VMEM((CH,), jnp.int32),
            pltpu.VMEM((CH,), jnp.int32),
            pltpu.VMEM((CH,), jnp.int32),
            pltpu.VMEM((CH, FB), F32),
            pltpu.VMEM_SHARED((NR, FB), F32),
            pltpu.VMEM_SHARED((NR, FB), F32),
            pltpu.SemaphoreType.DMA,
        ],
    )(_sc_base_body)
    return fn(gp, gn, ep, en, z80)


def _combine_body(ap_ref, an_ref, ys_ref, bp_ref, bn_ref,
                  hcat_ref, invp_ref, invn_ref):
    ap = ap_ref[0] + ap_ref[1]
    an = an_ref[0] + an_ref[1]
    cp = ap[:, H:H + 1]
    cn = an[:, H:H + 1]
    hp = ap[:, :H] / jnp.maximum(cp, 1.0) + ys_ref[:, :H] + bp_ref[...]
    hn = an[:, :H] / jnp.maximum(cn, 1.0) + ys_ref[:, H:] + bn_ref[...]
    hp = jnp.tanh(_normalize_rows(hp))
    hn = jnp.tanh(_normalize_rows(hn))
    hcat_ref[...] = jnp.concatenate([hp, hn], axis=1)
    invp_ref[...] = jnp.broadcast_to(1.0 / (cp + 1.0), (BM, FD))
    invn_ref[...] = jnp.broadcast_to(1.0 / (cn + 1.0), (BM, FD))


def _combine(accp, accn, ys, bpb, bnb):
    grid = (N // BM,)
    return pl.pallas_call(
        _combine_body,
        grid=grid,
        in_specs=[
            pl.BlockSpec((NC, BM, FB), lambda i: (0, i, 0)),
            pl.BlockSpec((NC, BM, FB), lambda i: (0, i, 0)),
            pl.BlockSpec((BM, 2 * H), lambda i: (i, 0)),
            pl.BlockSpec((1, H), lambda i: (0, 0)),
            pl.BlockSpec((1, H), lambda i: (0, 0)),
        ],
        out_specs=[
            pl.BlockSpec((BM, FD), lambda i: (i, 0)),
            pl.BlockSpec((BM, FD), lambda i: (i, 0)),
            pl.BlockSpec((BM, FD), lambda i: (i, 0)),
        ],
        out_shape=[
            jax.ShapeDtypeStruct((N, FD), F32),
            jax.ShapeDtypeStruct((N, FD), F32),
            jax.ShapeDtypeStruct((N, FD), F32),
        ],
    )(accp, accn, ys, bpb.reshape(1, H), bnb.reshape(1, H))


def _sc_deep_body(hcat, ep, en, radj_in, z128, tp_out, tn_out,
                  idxc, radj, gbuf, accp, accn, sem):
    c = lax.axis_index("c")
    s = lax.axis_index("s")
    wid = s * NC + c
    pltpu.sync_copy(z128, accp.at[pl.ds(s * ZR, ZR)])
    pltpu.sync_copy(z128, accn.at[pl.ds(s * ZR, ZR)])
    plsc.subcore_barrier()

    for eref, acc, setid in ((ep, accp, 0), (en, accn, 1)):
        ebase = wid * EPW

        def body(k, carry, eref=eref, acc=acc, setid=setid, ebase=ebase):
            base = ebase + k * CH
            pltpu.sync_copy(radj_in.at[setid, pl.ds(base, CH)], radj)
            pltpu.sync_copy(eref.at[1, pl.ds(base, CH)], idxc)
            pltpu.async_copy(hcat.at[idxc], gbuf, sem).wait()
            pltpu.sync_copy(gbuf, acc.at[radj], add=True)
            return carry

        lax.fori_loop(0, NCHUNK, body, 0)

    plsc.subcore_barrier()
    pltpu.sync_copy(accp.at[pl.ds(s * WR, WR)],
                    tp_out.at[c, pl.ds(s * WR, WR)])
    pltpu.sync_copy(accn.at[pl.ds(s * WR, WR)],
                    tn_out.at[c, pl.ds(s * WR, WR)])


def _sc_deep(hcat, ep, en, radj):
    z128 = jnp.zeros((ZR, FD), F32)
    mesh = plsc.VectorSubcoreMesh(core_axis_name="c", subcore_axis_name="s")
    fn = functools.partial(
        pl.kernel,
        mesh=mesh,
        out_type=[
            jax.ShapeDtypeStruct((NC, N, FD), F32),
            jax.ShapeDtypeStruct((NC, N, FD), F32),
        ],
        scratch_types=[
            pltpu.VMEM((CH,), jnp.int32),
            pltpu.VMEM((CH,), jnp.int32),
            pltpu.VMEM((CH, FD), F32),
            pltpu.VMEM_SHARED((NR, FD), F32),
            pltpu.VMEM_SHARED((NR, FD), F32),
            pltpu.SemaphoreType.DMA,
        ],
    )(_sc_deep_body)
    return fn(hcat, ep, en, radj, z128)


def _deep_combine_body(tp_ref, tn_ref, hcat_ref, invp_ref, invn_ref,
                       wp_ref, wn_ref, bp_ref, bn_ref, xmol_ref):
    hcat = hcat_ref[...]
    up = (tp_ref[0] + tp_ref[1] + hcat) * invp_ref[...]
    un = (tn_ref[0] + tn_ref[1] + hcat) * invn_ref[...]
    hp0 = hcat[:, :H]
    hn0 = hcat[:, H:]
    catp = jnp.concatenate([up[:, :H], un[:, H:], hp0], axis=1)
    catn = jnp.concatenate([up[:, H:], un[:, :H], hn0], axis=1)
    hp_pre = jnp.dot(catp, wp_ref[...], preferred_element_type=F32) + bp_ref[...]
    hn_pre = jnp.dot(catn, wn_ref[...], preferred_element_type=F32) + bn_ref[...]
    hp1 = jnp.tanh(_normalize_rows(hp_pre))
    hn1 = jnp.tanh(_normalize_rows(hn_pre))
    xmol_ref[...] = _normalize_rows(jnp.concatenate([hp1, hn1], axis=1))


def _deep_combine(tp, tn, hcat, invp, invn, Wpd, Wnd, bpd, bnd):
    grid = (N // BM,)
    return pl.pallas_call(
        _deep_combine_body,
        grid=grid,
        in_specs=[
            pl.BlockSpec((NC, BM, FD), lambda i: (0, i, 0)),
            pl.BlockSpec((NC, BM, FD), lambda i: (0, i, 0)),
            pl.BlockSpec((BM, FD), lambda i: (i, 0)),
            pl.BlockSpec((BM, FD), lambda i: (i, 0)),
            pl.BlockSpec((BM, FD), lambda i: (i, 0)),
            pl.BlockSpec((3 * H, H), lambda i: (0, 0)),
            pl.BlockSpec((3 * H, H), lambda i: (0, 0)),
            pl.BlockSpec((1, H), lambda i: (0, 0)),
            pl.BlockSpec((1, H), lambda i: (0, 0)),
        ],
        out_specs=pl.BlockSpec((BM, FD), lambda i: (i, 0)),
        out_shape=jax.ShapeDtypeStruct((N, FD), F32),
    )(tp, tn, hcat, invp, invn, Wpd, Wnd,
      bpd.reshape(1, H), bnd.reshape(1, H))


BP = 256
GN_ = N // BP


def _pred_body(xi_ref, xj_ref, mask_ref, lab_ref, pred_ref, loss_ref):
    i = pl.program_id(0)
    j = pl.program_id(1)

    @pl.when(jnp.logical_and(i == 0, j == 0))
    def _init():
        loss_ref[...] = jnp.zeros((1, 1), F32)

    b = lax.dot_general(xi_ref[...], xj_ref[...],
                        (((1,), (1,)), ((), ())),
                        preferred_element_type=F32) * mask_ref[...]
    pred_ref[...] = b
    r = b - lab_ref[...]
    loss_ref[...] += jnp.sum(r * r).reshape(1, 1)

    @pl.when(jnp.logical_and(i == GN_ - 1, j == GN_ - 1))
    def _fin():
        loss_ref[...] = loss_ref[...] * (1.0 / float(N * N))


def _pred_loss(xmol, label_mask, labels2d):
    grid = (GN_, GN_)
    return pl.pallas_call(
        _pred_body,
        grid=grid,
        in_specs=[
            pl.BlockSpec((BP, FD), lambda i, j: (i, 0)),
            pl.BlockSpec((BP, FD), lambda i, j: (j, 0)),
            pl.BlockSpec((BP, BP), lambda i, j: (i, j)),
            pl.BlockSpec((BP, BP), lambda i, j: (i, j)),
        ],
        out_specs=[
            pl.BlockSpec((BP, BP), lambda i, j: (i, j)),
            pl.BlockSpec((1, 1), lambda i, j: (0, 0)),
        ],
        out_shape=[
            jax.ShapeDtypeStruct((N, N), F32),
            jax.ShapeDtypeStruct((1, 1), F32),
        ],
    )(xmol, xmol, label_mask, labels2d)


def kernel(X, positive_edges, negative_edges, labels, label_mask,
           Wpb, bpb, Wnb, bnb, Wpd, bpd, Wnd, bnd):
    ep = positive_edges.astype(jnp.int32)
    en = negative_edges.astype(jnp.int32)
    Wcat = jnp.concatenate([Wpb[:D], Wnb[:D], Wpb[D:], Wnb[D:]], axis=1)

    gp, gn, ys = _project(X, Wcat)
    accp, accn, radj = _sc_base(gp, gn, ep, en)
    hcat, invp, invn = _combine(accp, accn, ys, bpb, bnb)
    tp, tn = _sc_deep(hcat, ep, en, radj)
    xmol = _deep_combine(tp, tn, hcat, invp, invn, Wpd, Wnd, bpd, bnd)
    pred2, lossm = _pred_loss(xmol, label_mask, labels.reshape(N, N))
    return (lossm[0, 0], xmol, pred2.reshape(-1))


# speedup vs baseline: 10.8369x; 1.3378x over previous
import functools

import jax
import jax.numpy as jnp
from jax import lax
from jax.experimental import pallas as pl
from jax.experimental.pallas import tpu as pltpu
from jax.experimental.pallas import tpu_sc as plsc

N = 4096
D = 2048
E = 65536
H = 64

F32 = jnp.float32

NC, NS = 2, 16
NW = NC * NS
CH = 128
EPW = E // NW
NCHUNK = EPW // CH
TRASH = N
NR = N + 128
ZR = NR // NS
WR = N // NS
FB = 128
FD = 128

BM = 256


def _normalize_rows(x):
    n = jnp.sqrt(jnp.sum(x * x, axis=1, keepdims=True))
    return x / jnp.maximum(n, 1e-12)


def _project_body(x_ref, w_ref, gp_ref, gn_ref, ys_ref):
    p = jnp.dot(x_ref[...], w_ref[...], preferred_element_type=F32)
    ones = jnp.ones((BM, FB - H), F32)
    gp_ref[...] = jnp.concatenate([p[:, :H], ones], axis=1)
    gn_ref[...] = jnp.concatenate([p[:, H:2 * H], ones], axis=1)
    ys_ref[...] = p[:, 2 * H:]


def _project(X, Wcat):
    grid = (N // BM,)
    return pl.pallas_call(
        _project_body,
        grid=grid,
        in_specs=[
            pl.BlockSpec((BM, D), lambda i: (i, 0)),
            pl.BlockSpec((D, 4 * H), lambda i: (0, 0)),
        ],
        out_specs=[
            pl.BlockSpec((BM, FB), lambda i: (i, 0)),
            pl.BlockSpec((BM, FB), lambda i: (i, 0)),
            pl.BlockSpec((BM, 2 * H), lambda i: (i, 0)),
        ],
        out_shape=[
            jax.ShapeDtypeStruct((N, FB), F32),
            jax.ShapeDtypeStruct((N, FB), F32),
            jax.ShapeDtypeStruct((N, 2 * H), F32),
        ],
    )(X, Wcat)


def _sc_base_body(gp, gn, ep, en, z80, accp_out, accn_out, radj_out,
                  idxr, idxc, radj, gbuf, accp, accn, sem):
    c = lax.axis_index("c")
    s = lax.axis_index("s")
    wid = s * NC + c
    pltpu.sync_copy(z80, accp.at[pl.ds(s * ZR, ZR)])
    pltpu.sync_copy(z80, accn.at[pl.ds(s * ZR, ZR)])
    plsc.subcore_barrier()

    for eref, gref, acc, setid in ((ep, gp, accp, 0), (en, gn, accn, 1)):
        ebase = wid * EPW

        def body(k, carry, eref=eref, gref=gref, acc=acc, setid=setid,
                 ebase=ebase):
            base = ebase + k * CH
            pltpu.sync_copy(eref.at[0, pl.ds(base, CH)], idxr)
            pltpu.sync_copy(eref.at[1, pl.ds(base, CH)], idxc)

            def adj(j, carry2):
                r = idxr[pl.ds(j * 16, 16)]
                cc = idxc[pl.ds(j * 16, 16)]
                radj[pl.ds(j * 16, 16)] = jnp.where(r == cc, TRASH, r)
                return carry2

            lax.fori_loop(0, CH // 16, adj, 0)
            pltpu.async_copy(gref.at[idxc], gbuf, sem).wait()
            pltpu.sync_copy(gbuf, acc.at[radj], add=True)
            pltpu.sync_copy(radj, radj_out.at[setid, pl.ds(base, CH)])
            return carry

        lax.fori_loop(0, NCHUNK, body, 0)

    plsc.subcore_barrier()
    pltpu.sync_copy(accp.at[pl.ds(s * WR, WR)],
                    accp_out.at[c, pl.ds(s * WR, WR)])
    pltpu.sync_copy(accn.at[pl.ds(s * WR, WR)],
                    accn_out.at[c, pl.ds(s * WR, WR)])


def _sc_base(gp, gn, ep, en):
    z80 = jnp.zeros((ZR, FB), F32)
    mesh = plsc.VectorSubcoreMesh(core_axis_name="c", subcore_axis_name="s")
    fn = functools.partial(
        pl.kernel,
        mesh=mesh,
        out_type=[
            jax.ShapeDtypeStruct((NC, N, FB), F32),
            jax.ShapeDtypeStruct((NC, N, FB), F32),
            jax.ShapeDtypeStruct((2, E), jnp.int32),
        ],
        scratch_types=[
            pltpu.VMEM((CH,), jnp.int32),
            pltpu.VMEM((CH,), jnp.int32),
            pltpu.VMEM((CH,), jnp.int32),
            pltpu.VMEM((CH, FB), F32),
            pltpu.VMEM_SHARED((NR, FB), F32),
            pltpu.VMEM_SHARED((NR, FB), F32),
            pltpu.SemaphoreType.DMA,
        ],
    )(_sc_base_body)
    return fn(gp, gn, ep, en, z80)


def _combine_body(ap_ref, an_ref, ys_ref, bp_ref, bn_ref,
                  hcat_ref, invp_ref, invn_ref):
    ap = ap_ref[0] + ap_ref[1]
    an = an_ref[0] + an_ref[1]
    cp = ap[:, H:H + 1]
    cn = an[:, H:H + 1]
    hp = ap[:, :H] / jnp.maximum(cp, 1.0) + ys_ref[:, :H] + bp_ref[...]
    hn = an[:, :H] / jnp.maximum(cn, 1.0) + ys_ref[:, H:] + bn_ref[...]
    hp = jnp.tanh(_normalize_rows(hp))
    hn = jnp.tanh(_normalize_rows(hn))
    hcat_ref[...] = jnp.concatenate([hp, hn], axis=1)
    invp_ref[...] = jnp.broadcast_to(1.0 / (cp + 1.0), (BM, FD))
    invn_ref[...] = jnp.broadcast_to(1.0 / (cn + 1.0), (BM, FD))


def _combine(accp, accn, ys, bpb, bnb):
    grid = (N // BM,)
    return pl.pallas_call(
        _combine_body,
        grid=grid,
        in_specs=[
            pl.BlockSpec((NC, BM, FB), lambda i: (0, i, 0)),
            pl.BlockSpec((NC, BM, FB), lambda i: (0, i, 0)),
            pl.BlockSpec((BM, 2 * H), lambda i: (i, 0)),
            pl.BlockSpec((1, H), lambda i: (0, 0)),
            pl.BlockSpec((1, H), lambda i: (0, 0)),
        ],
        out_specs=[
            pl.BlockSpec((BM, FD), lambda i: (i, 0)),
            pl.BlockSpec((BM, FD), lambda i: (i, 0)),
            pl.BlockSpec((BM, FD), lambda i: (i, 0)),
        ],
        out_shape=[
            jax.ShapeDtypeStruct((N, FD), F32),
            jax.ShapeDtypeStruct((N, FD), F32),
            jax.ShapeDtypeStruct((N, FD), F32),
        ],
    )(accp, accn, ys, bpb.reshape(1, H), bnb.reshape(1, H))


def _sc_deep_body(hcat, ep, en, radj_in, z128, tp_out, tn_out,
                  idxc, radj, gbuf, accp, accn, sem):
    c = lax.axis_index("c")
    s = lax.axis_index("s")
    wid = s * NC + c
    pltpu.sync_copy(z128, accp.at[pl.ds(s * ZR, ZR)])
    pltpu.sync_copy(z128, accn.at[pl.ds(s * ZR, ZR)])
    plsc.subcore_barrier()

    for eref, acc, setid in ((ep, accp, 0), (en, accn, 1)):
        ebase = wid * EPW

        def body(k, carry, eref=eref, acc=acc, setid=setid, ebase=ebase):
            base = ebase + k * CH
            pltpu.sync_copy(radj_in.at[setid, pl.ds(base, CH)], radj)
            pltpu.sync_copy(eref.at[1, pl.ds(base, CH)], idxc)
            pltpu.async_copy(hcat.at[idxc], gbuf, sem).wait()
            pltpu.sync_copy(gbuf, acc.at[radj], add=True)
            return carry

        lax.fori_loop(0, NCHUNK, body, 0)

    plsc.subcore_barrier()
    pltpu.sync_copy(accp.at[pl.ds(s * WR, WR)],
                    tp_out.at[c, pl.ds(s * WR, WR)])
    pltpu.sync_copy(accn.at[pl.ds(s * WR, WR)],
                    tn_out.at[c, pl.ds(s * WR, WR)])


def _sc_deep(hcat, ep, en, radj):
    z128 = jnp.zeros((ZR, FD), F32)
    mesh = plsc.VectorSubcoreMesh(core_axis_name="c", subcore_axis_name="s")
    fn = functools.partial(
        pl.kernel,
        mesh=mesh,
        out_type=[
            jax.ShapeDtypeStruct((NC, N, FD), F32),
            jax.ShapeDtypeStruct((NC, N, FD), F32),
        ],
        scratch_types=[
            pltpu.VMEM((CH,), jnp.int32),
            pltpu.VMEM((CH,), jnp.int32),
            pltpu.VMEM((CH, FD), F32),
            pltpu.VMEM_SHARED((NR, FD), F32),
            pltpu.VMEM_SHARED((NR, FD), F32),
            pltpu.SemaphoreType.DMA,
        ],
    )(_sc_deep_body)
    return fn(hcat, ep, en, radj, z128)


def _deep_combine_body(tp_ref, tn_ref, hcat_ref, invp_ref, invn_ref,
                       wp_ref, wn_ref, bp_ref, bn_ref, xmol_ref):
    hcat = hcat_ref[...]
    up = (tp_ref[0] + tp_ref[1] + hcat) * invp_ref[...]
    un = (tn_ref[0] + tn_ref[1] + hcat) * invn_ref[...]
    hp0 = hcat[:, :H]
    hn0 = hcat[:, H:]
    catp = jnp.concatenate([up[:, :H], un[:, H:], hp0], axis=1)
    catn = jnp.concatenate([up[:, H:], un[:, :H], hn0], axis=1)
    hp_pre = jnp.dot(catp, wp_ref[...], preferred_element_type=F32) + bp_ref[...]
    hn_pre = jnp.dot(catn, wn_ref[...], preferred_element_type=F32) + bn_ref[...]
    hp1 = jnp.tanh(_normalize_rows(hp_pre))
    hn1 = jnp.tanh(_normalize_rows(hn_pre))
    xmol_ref[...] = _normalize_rows(jnp.concatenate([hp1, hn1], axis=1))


def _deep_combine(tp, tn, hcat, invp, invn, Wpd, Wnd, bpd, bnd):
    grid = (N // BM,)
    return pl.pallas_call(
        _deep_combine_body,
        grid=grid,
        in_specs=[
            pl.BlockSpec((NC, BM, FD), lambda i: (0, i, 0)),
            pl.BlockSpec((NC, BM, FD), lambda i: (0, i, 0)),
            pl.BlockSpec((BM, FD), lambda i: (i, 0)),
            pl.BlockSpec((BM, FD), lambda i: (i, 0)),
            pl.BlockSpec((BM, FD), lambda i: (i, 0)),
            pl.BlockSpec((3 * H, H), lambda i: (0, 0)),
            pl.BlockSpec((3 * H, H), lambda i: (0, 0)),
            pl.BlockSpec((1, H), lambda i: (0, 0)),
            pl.BlockSpec((1, H), lambda i: (0, 0)),
        ],
        out_specs=pl.BlockSpec((BM, FD), lambda i: (i, 0)),
        out_shape=jax.ShapeDtypeStruct((N, FD), F32),
    )(tp, tn, hcat, invp, invn, Wpd, Wnd,
      bpd.reshape(1, H), bnd.reshape(1, H))


BP = 256
GN_ = N // BP


def _pred_body(xi_ref, xall_ref, mask_ref, lab_ref, pred_ref, loss_ref):
    i = pl.program_id(0)

    @pl.when(i == 0)
    def _init():
        loss_ref[...] = jnp.zeros((1, 1), F32)

    b = lax.dot_general(xi_ref[...], xall_ref[...],
                        (((1,), (1,)), ((), ())),
                        preferred_element_type=F32) * mask_ref[...]
    pred_ref[...] = b
    r = b - lab_ref[...]
    loss_ref[...] += jnp.sum(r * r).reshape(1, 1)

    @pl.when(i == GN_ - 1)
    def _fin():
        loss_ref[...] = loss_ref[...] * (1.0 / float(N * N))


def _pred_loss(xmol, label_mask, labels2d):
    grid = (GN_,)
    return pl.pallas_call(
        _pred_body,
        grid=grid,
        in_specs=[
            pl.BlockSpec((BP, FD), lambda i: (i, 0)),
            pl.BlockSpec((N, FD), lambda i: (0, 0)),
            pl.BlockSpec((BP, N), lambda i: (i, 0)),
            pl.BlockSpec((BP, N), lambda i: (i, 0)),
        ],
        out_specs=[
            pl.BlockSpec((BP, N), lambda i: (i, 0)),
            pl.BlockSpec((1, 1), lambda i: (0, 0)),
        ],
        out_shape=[
            jax.ShapeDtypeStruct((N, N), F32),
            jax.ShapeDtypeStruct((1, 1), F32),
        ],
    )(xmol, xmol, label_mask, labels2d)


def kernel(X, positive_edges, negative_edges, labels, label_mask,
           Wpb, bpb, Wnb, bnb, Wpd, bpd, Wnd, bnd):
    ep = positive_edges.astype(jnp.int32)
    en = negative_edges.astype(jnp.int32)
    Wcat = jnp.concatenate([Wpb[:D], Wnb[:D], Wpb[D:], Wnb[D:]], axis=1)

    gp, gn, ys = _project(X, Wcat)
    accp, accn, radj = _sc_base(gp, gn, ep, en)
    hcat, invp, invn = _combine(accp, accn, ys, bpb, bnb)
    tp, tn = _sc_deep(hcat, ep, en, radj)
    xmol = _deep_combine(tp, tn, hcat, invp, invn, Wpd, Wnd, bpd, bnd)
    pred2, lossm = _pred_loss(xmol, label_mask, labels.reshape(N, N))
    return (lossm[0, 0], xmol, pred2.reshape(-1))


# speedup vs baseline: 13.8171x; 1.2750x over previous
import functools

import jax
import jax.numpy as jnp
from jax import lax
from jax.experimental import pallas as pl
from jax.experimental.pallas import tpu as pltpu
from jax.experimental.pallas import tpu_sc as plsc

N = 4096
D = 2048
E = 65536
H = 64

F32 = jnp.float32

NC, NS = 2, 16
NW = NC * NS
CH = 128
EPW = E // NW
NCHUNK = EPW // CH
TRASH = N
NR = N + 128
ZR = NR // NS
WR = N // NS
FB = 128
FD = 128

BM = 256


def _normalize_rows(x):
    n = jnp.sqrt(jnp.sum(x * x, axis=1, keepdims=True))
    return x / jnp.maximum(n, 1e-12)


def _project_body(x_ref, w_ref, gp_ref, gn_ref, ys_ref):
    p = jnp.dot(x_ref[...], w_ref[...], preferred_element_type=F32)
    ones = jnp.ones((BM, FB - H), F32)
    gp_ref[...] = jnp.concatenate([p[:, :H], ones], axis=1)
    gn_ref[...] = jnp.concatenate([p[:, H:2 * H], ones], axis=1)
    ys_ref[...] = p[:, 2 * H:]


def _project(X, Wcat):
    grid = (N // BM,)
    return pl.pallas_call(
        _project_body,
        grid=grid,
        in_specs=[
            pl.BlockSpec((BM, D), lambda i: (i, 0)),
            pl.BlockSpec((D, 4 * H), lambda i: (0, 0)),
        ],
        out_specs=[
            pl.BlockSpec((BM, FB), lambda i: (i, 0)),
            pl.BlockSpec((BM, FB), lambda i: (i, 0)),
            pl.BlockSpec((BM, 2 * H), lambda i: (i, 0)),
        ],
        out_shape=[
            jax.ShapeDtypeStruct((N, FB), F32),
            jax.ShapeDtypeStruct((N, FB), F32),
            jax.ShapeDtypeStruct((N, 2 * H), F32),
        ],
    )(X, Wcat)


NB = 2
NG = NCHUNK // NB


def _seg_sum_set(eref, gref, acc, wid, idx2, radja, gbuf, gsem):
    ebase = wid * EPW
    pltpu.sync_copy(eref.at[:, pl.ds(ebase, EPW)], idx2)
    for b in range(NB):
        pltpu.async_copy(gref.at[idx2.at[1, pl.ds(b * CH, CH)]],
                         gbuf.at[b], gsem.at[b])

    def adj(j, carry):
        r = idx2[0, pl.ds(j * 16, 16)]
        cc = idx2[1, pl.ds(j * 16, 16)]
        radja[pl.ds(j * 16, 16)] = jnp.where(r == cc, TRASH, r)
        return carry

    lax.fori_loop(0, EPW // 16, adj, 0)

    def ring(g, carry):
        for b in range(NB):
            k = g * NB + b
            pltpu.make_async_copy(
                gref.at[idx2.at[1, pl.ds(k * CH, CH)]],
                gbuf.at[b], gsem.at[b]).wait()
            pltpu.sync_copy(gbuf.at[b],
                            acc.at[radja.at[pl.ds(k * CH, CH)]], add=True)

            @pl.when(k + NB < NCHUNK)
            def _issue(k=k, b=b):
                pltpu.async_copy(
                    gref.at[idx2.at[1, pl.ds((k + NB) * CH, CH)]],
                    gbuf.at[b], gsem.at[b])
        return carry

    lax.fori_loop(0, NG, ring, 0)


def _sc_base_body(gp, gn, ep, en, z80, accp_out, accn_out,
                  idx2, radja, gbuf, accp, accn, gsem):
    c = lax.axis_index("c")
    s = lax.axis_index("s")
    wid = s * NC + c
    pltpu.sync_copy(z80, accp.at[pl.ds(s * ZR, ZR)])
    pltpu.sync_copy(z80, accn.at[pl.ds(s * ZR, ZR)])
    plsc.subcore_barrier()

    _seg_sum_set(ep, gp, accp, wid, idx2, radja, gbuf, gsem)
    _seg_sum_set(en, gn, accn, wid, idx2, radja, gbuf, gsem)

    plsc.subcore_barrier()
    pltpu.sync_copy(accp.at[pl.ds(s * WR, WR)],
                    accp_out.at[c, pl.ds(s * WR, WR)])
    pltpu.sync_copy(accn.at[pl.ds(s * WR, WR)],
                    accn_out.at[c, pl.ds(s * WR, WR)])


def _sc_base(gp, gn, ep, en):
    z80 = jnp.zeros((ZR, FB), F32)
    mesh = plsc.VectorSubcoreMesh(core_axis_name="c", subcore_axis_name="s")
    fn = functools.partial(
        pl.kernel,
        mesh=mesh,
        out_type=[
            jax.ShapeDtypeStruct((NC, N, FB), F32),
            jax.ShapeDtypeStruct((NC, N, FB), F32),
        ],
        scratch_types=[
            pltpu.VMEM((2, EPW), jnp.int32),
            pltpu.VMEM((EPW,), jnp.int32),
            pltpu.VMEM((NB, CH, FB), F32),
            pltpu.VMEM_SHARED((NR, FB), F32),
            pltpu.VMEM_SHARED((NR, FB), F32),
            pltpu.SemaphoreType.DMA((NB,)),
        ],
    )(_sc_base_body)
    return fn(gp, gn, ep, en, z80)


def _combine_body(ap_ref, an_ref, ys_ref, bp_ref, bn_ref,
                  hcat_ref, invp_ref, invn_ref):
    ap = ap_ref[0] + ap_ref[1]
    an = an_ref[0] + an_ref[1]
    cp = ap[:, H:H + 1]
    cn = an[:, H:H + 1]
    hp = ap[:, :H] / jnp.maximum(cp, 1.0) + ys_ref[:, :H] + bp_ref[...]
    hn = an[:, :H] / jnp.maximum(cn, 1.0) + ys_ref[:, H:] + bn_ref[...]
    hp = jnp.tanh(_normalize_rows(hp))
    hn = jnp.tanh(_normalize_rows(hn))
    hcat_ref[...] = jnp.concatenate([hp, hn], axis=1)
    invp_ref[...] = jnp.broadcast_to(1.0 / (cp + 1.0), (BM, FD))
    invn_ref[...] = jnp.broadcast_to(1.0 / (cn + 1.0), (BM, FD))


def _combine(accp, accn, ys, bpb, bnb):
    grid = (N // BM,)
    return pl.pallas_call(
        _combine_body,
        grid=grid,
        in_specs=[
            pl.BlockSpec((NC, BM, FB), lambda i: (0, i, 0)),
            pl.BlockSpec((NC, BM, FB), lambda i: (0, i, 0)),
            pl.BlockSpec((BM, 2 * H), lambda i: (i, 0)),
            pl.BlockSpec((1, H), lambda i: (0, 0)),
            pl.BlockSpec((1, H), lambda i: (0, 0)),
        ],
        out_specs=[
            pl.BlockSpec((BM, FD), lambda i: (i, 0)),
            pl.BlockSpec((BM, FD), lambda i: (i, 0)),
            pl.BlockSpec((BM, FD), lambda i: (i, 0)),
        ],
        out_shape=[
            jax.ShapeDtypeStruct((N, FD), F32),
            jax.ShapeDtypeStruct((N, FD), F32),
            jax.ShapeDtypeStruct((N, FD), F32),
        ],
    )(accp, accn, ys, bpb.reshape(1, H), bnb.reshape(1, H))


def _sc_deep_body(hcat, ep, en, z128, tp_out, tn_out,
                  idx2, radja, gbuf, accp, accn, gsem):
    c = lax.axis_index("c")
    s = lax.axis_index("s")
    wid = s * NC + c
    pltpu.sync_copy(z128, accp.at[pl.ds(s * ZR, ZR)])
    pltpu.sync_copy(z128, accn.at[pl.ds(s * ZR, ZR)])
    plsc.subcore_barrier()

    _seg_sum_set(ep, hcat, accp, wid, idx2, radja, gbuf, gsem)
    _seg_sum_set(en, hcat, accn, wid, idx2, radja, gbuf, gsem)

    plsc.subcore_barrier()
    pltpu.sync_copy(accp.at[pl.ds(s * WR, WR)],
                    tp_out.at[c, pl.ds(s * WR, WR)])
    pltpu.sync_copy(accn.at[pl.ds(s * WR, WR)],
                    tn_out.at[c, pl.ds(s * WR, WR)])


def _sc_deep(hcat, ep, en):
    z128 = jnp.zeros((ZR, FD), F32)
    mesh = plsc.VectorSubcoreMesh(core_axis_name="c", subcore_axis_name="s")
    fn = functools.partial(
        pl.kernel,
        mesh=mesh,
        out_type=[
            jax.ShapeDtypeStruct((NC, N, FD), F32),
            jax.ShapeDtypeStruct((NC, N, FD), F32),
        ],
        scratch_types=[
            pltpu.VMEM((2, EPW), jnp.int32),
            pltpu.VMEM((EPW,), jnp.int32),
            pltpu.VMEM((NB, CH, FD), F32),
            pltpu.VMEM_SHARED((NR, FD), F32),
            pltpu.VMEM_SHARED((NR, FD), F32),
            pltpu.SemaphoreType.DMA((NB,)),
        ],
    )(_sc_deep_body)
    return fn(hcat, ep, en, z128)


def _deep_combine_body(tp_ref, tn_ref, hcat_ref, invp_ref, invn_ref,
                       wp_ref, wn_ref, bp_ref, bn_ref, xmol_ref):
    hcat = hcat_ref[...]
    up = (tp_ref[0] + tp_ref[1] + hcat) * invp_ref[...]
    un = (tn_ref[0] + tn_ref[1] + hcat) * invn_ref[...]
    hp0 = hcat[:, :H]
    hn0 = hcat[:, H:]
    catp = jnp.concatenate([up[:, :H], un[:, H:], hp0], axis=1)
    catn = jnp.concatenate([up[:, H:], un[:, :H], hn0], axis=1)
    hp_pre = jnp.dot(catp, wp_ref[...], preferred_element_type=F32) + bp_ref[...]
    hn_pre = jnp.dot(catn, wn_ref[...], preferred_element_type=F32) + bn_ref[...]
    hp1 = jnp.tanh(_normalize_rows(hp_pre))
    hn1 = jnp.tanh(_normalize_rows(hn_pre))
    xmol_ref[...] = _normalize_rows(jnp.concatenate([hp1, hn1], axis=1))


def _deep_combine(tp, tn, hcat, invp, invn, Wpd, Wnd, bpd, bnd):
    grid = (N // BM,)
    return pl.pallas_call(
        _deep_combine_body,
        grid=grid,
        in_specs=[
            pl.BlockSpec((NC, BM, FD), lambda i: (0, i, 0)),
            pl.BlockSpec((NC, BM, FD), lambda i: (0, i, 0)),
            pl.BlockSpec((BM, FD), lambda i: (i, 0)),
            pl.BlockSpec((BM, FD), lambda i: (i, 0)),
            pl.BlockSpec((BM, FD), lambda i: (i, 0)),
            pl.BlockSpec((3 * H, H), lambda i: (0, 0)),
            pl.BlockSpec((3 * H, H), lambda i: (0, 0)),
            pl.BlockSpec((1, H), lambda i: (0, 0)),
            pl.BlockSpec((1, H), lambda i: (0, 0)),
        ],
        out_specs=pl.BlockSpec((BM, FD), lambda i: (i, 0)),
        out_shape=jax.ShapeDtypeStruct((N, FD), F32),
    )(tp, tn, hcat, invp, invn, Wpd, Wnd,
      bpd.reshape(1, H), bnd.reshape(1, H))


BP = 256
GN_ = N // BP


def _pred_body(xi_ref, xall_ref, mask_ref, lab_ref, pred_ref, loss_ref):
    i = pl.program_id(0)

    @pl.when(i == 0)
    def _init():
        loss_ref[...] = jnp.zeros((1, 1), F32)

    b = lax.dot_general(xi_ref[...], xall_ref[...],
                        (((1,), (1,)), ((), ())),
                        preferred_element_type=F32) * mask_ref[...]
    pred_ref[...] = b
    r = b - lab_ref[...]
    loss_ref[...] += jnp.sum(r * r).reshape(1, 1)

    @pl.when(i == GN_ - 1)
    def _fin():
        loss_ref[...] = loss_ref[...] * (1.0 / float(N * N))


def _pred_loss(xmol, label_mask, labels2d):
    grid = (GN_,)
    return pl.pallas_call(
        _pred_body,
        grid=grid,
        in_specs=[
            pl.BlockSpec((BP, FD), lambda i: (i, 0)),
            pl.BlockSpec((N, FD), lambda i: (0, 0)),
            pl.BlockSpec((BP, N), lambda i: (i, 0)),
            pl.BlockSpec((BP, N), lambda i: (i, 0)),
        ],
        out_specs=[
            pl.BlockSpec((BP, N), lambda i: (i, 0)),
            pl.BlockSpec((1, 1), lambda i: (0, 0)),
        ],
        out_shape=[
            jax.ShapeDtypeStruct((N, N), F32),
            jax.ShapeDtypeStruct((1, 1), F32),
        ],
    )(xmol, xmol, label_mask, labels2d)


def kernel(X, positive_edges, negative_edges, labels, label_mask,
           Wpb, bpb, Wnb, bnb, Wpd, bpd, Wnd, bnd):
    ep = positive_edges.astype(jnp.int32)
    en = negative_edges.astype(jnp.int32)
    Wcat = jnp.concatenate([Wpb[:D], Wnb[:D], Wpb[D:], Wnb[D:]], axis=1)

    gp, gn, ys = _project(X, Wcat)
    accp, accn = _sc_base(gp, gn, ep, en)
    hcat, invp, invn = _combine(accp, accn, ys, bpb, bnb)
    tp, tn = _sc_deep(hcat, ep, en)
    xmol = _deep_combine(tp, tn, hcat, invp, invn, Wpd, Wnd, bpd, bnd)
    pred2, lossm = _pred_loss(xmol, label_mask, labels.reshape(N, N))
    return (lossm[0, 0], xmol, pred2.reshape(-1))


# speedup vs baseline: 17.6860x; 1.2800x over previous
import functools

import jax
import jax.numpy as jnp
from jax import lax
from jax.experimental import pallas as pl
from jax.experimental.pallas import tpu as pltpu
from jax.experimental.pallas import tpu_sc as plsc

N = 4096
D = 2048
E = 65536
H = 64

F32 = jnp.float32

NC, NS = 2, 16
NW = NC * NS
CH = 128
EPW = E // NW
NCHUNK = EPW // CH
TRASH = N
NR = N + 128
ZR = NR // NS
WR = N // NS
FB = 128
FD = 128

BM = 256


def _normalize_rows(x):
    n = jnp.sqrt(jnp.sum(x * x, axis=1, keepdims=True))
    return x / jnp.maximum(n, 1e-12)


def _project_body(x_ref, w_ref, gp_ref, gn_ref, ys_ref):
    p = jnp.dot(x_ref[...], w_ref[...], preferred_element_type=F32)
    ones = jnp.ones((BM, FB - H), F32)
    gp_ref[...] = jnp.concatenate([p[:, :H], ones], axis=1)
    gn_ref[...] = jnp.concatenate([p[:, H:2 * H], ones], axis=1)
    ys_ref[...] = p[:, 2 * H:]


def _project(X, Wcat):
    grid = (N // BM,)
    return pl.pallas_call(
        _project_body,
        grid=grid,
        in_specs=[
            pl.BlockSpec((BM, D), lambda i: (i, 0)),
            pl.BlockSpec((D, 4 * H), lambda i: (0, 0)),
        ],
        out_specs=[
            pl.BlockSpec((BM, FB), lambda i: (i, 0)),
            pl.BlockSpec((BM, FB), lambda i: (i, 0)),
            pl.BlockSpec((BM, 2 * H), lambda i: (i, 0)),
        ],
        out_shape=[
            jax.ShapeDtypeStruct((N, FB), F32),
            jax.ShapeDtypeStruct((N, FB), F32),
            jax.ShapeDtypeStruct((N, 2 * H), F32),
        ],
    )(X, Wcat)


NB = 2
NG = NCHUNK // NB


def _seg_sum_set(eref, gref, acc, wid, idx2, radja, gbuf, gsem):
    ebase = wid * EPW
    pltpu.sync_copy(eref.at[:, pl.ds(ebase, EPW)], idx2)
    for b in range(NB):
        pltpu.async_copy(gref.at[idx2.at[1, pl.ds(b * CH, CH)]],
                         gbuf.at[b], gsem.at[b])

    def adj(j, carry):
        r = idx2[0, pl.ds(j * 16, 16)]
        cc = idx2[1, pl.ds(j * 16, 16)]
        radja[pl.ds(j * 16, 16)] = jnp.where(r == cc, TRASH, r)
        return carry

    lax.fori_loop(0, EPW // 16, adj, 0)

    def ring(g, carry):
        for b in range(NB):
            k = g * NB + b
            pltpu.make_async_copy(
                gref.at[idx2.at[1, pl.ds(k * CH, CH)]],
                gbuf.at[b], gsem.at[b]).wait()
            pltpu.sync_copy(gbuf.at[b],
                            acc.at[radja.at[pl.ds(k * CH, CH)]], add=True)

            @pl.when(k + NB < NCHUNK)
            def _issue(k=k, b=b):
                pltpu.async_copy(
                    gref.at[idx2.at[1, pl.ds((k + NB) * CH, CH)]],
                    gbuf.at[b], gsem.at[b])
        return carry

    lax.fori_loop(0, NG, ring, 0)


def _sc_base_body(gp, gn, ep, en, z80, accp_out, accn_out,
                  idx2, radja, gbuf, accp, accn, gsem):
    c = lax.axis_index("c")
    s = lax.axis_index("s")
    wid = s * NC + c
    pltpu.sync_copy(z80, accp.at[pl.ds(s * ZR, ZR)])
    pltpu.sync_copy(z80, accn.at[pl.ds(s * ZR, ZR)])
    plsc.subcore_barrier()

    _seg_sum_set(ep, gp, accp, wid, idx2, radja, gbuf, gsem)
    _seg_sum_set(en, gn, accn, wid, idx2, radja, gbuf, gsem)

    plsc.subcore_barrier()
    pltpu.sync_copy(accp.at[pl.ds(s * WR, WR)],
                    accp_out.at[c, pl.ds(s * WR, WR)])
    pltpu.sync_copy(accn.at[pl.ds(s * WR, WR)],
                    accn_out.at[c, pl.ds(s * WR, WR)])


def _sc_base(gp, gn, ep, en):
    z80 = jnp.zeros((ZR, FB), F32)
    mesh = plsc.VectorSubcoreMesh(core_axis_name="c", subcore_axis_name="s")
    fn = functools.partial(
        pl.kernel,
        mesh=mesh,
        out_type=[
            jax.ShapeDtypeStruct((NC, N, FB), F32),
            jax.ShapeDtypeStruct((NC, N, FB), F32),
        ],
        scratch_types=[
            pltpu.VMEM((2, EPW), jnp.int32),
            pltpu.VMEM((EPW,), jnp.int32),
            pltpu.VMEM((NB, CH, FB), F32),
            pltpu.VMEM_SHARED((NR, FB), F32),
            pltpu.VMEM_SHARED((NR, FB), F32),
            pltpu.SemaphoreType.DMA((NB,)),
        ],
    )(_sc_base_body)
    return fn(gp, gn, ep, en, z80)


def _combine_body(ap_ref, an_ref, ys_ref, bp_ref, bn_ref,
                  hcat_ref, invp_ref, invn_ref):
    ap = ap_ref[0] + ap_ref[1]
    an = an_ref[0] + an_ref[1]
    cp = ap[:, H:H + 1]
    cn = an[:, H:H + 1]
    hp = ap[:, :H] / jnp.maximum(cp, 1.0) + ys_ref[:, :H] + bp_ref[...]
    hn = an[:, :H] / jnp.maximum(cn, 1.0) + ys_ref[:, H:] + bn_ref[...]
    hp = jnp.tanh(_normalize_rows(hp))
    hn = jnp.tanh(_normalize_rows(hn))
    hcat_ref[...] = jnp.concatenate([hp, hn], axis=1)
    invp_ref[...] = jnp.broadcast_to(1.0 / (cp + 1.0), (BM, FD))
    invn_ref[...] = jnp.broadcast_to(1.0 / (cn + 1.0), (BM, FD))


def _combine(accp, accn, ys, bpb, bnb):
    grid = (N // BM,)
    return pl.pallas_call(
        _combine_body,
        grid=grid,
        in_specs=[
            pl.BlockSpec((NC, BM, FB), lambda i: (0, i, 0)),
            pl.BlockSpec((NC, BM, FB), lambda i: (0, i, 0)),
            pl.BlockSpec((BM, 2 * H), lambda i: (i, 0)),
            pl.BlockSpec((1, H), lambda i: (0, 0)),
            pl.BlockSpec((1, H), lambda i: (0, 0)),
        ],
        out_specs=[
            pl.BlockSpec((BM, FD), lambda i: (i, 0)),
            pl.BlockSpec((BM, FD), lambda i: (i, 0)),
            pl.BlockSpec((BM, FD), lambda i: (i, 0)),
        ],
        out_shape=[
            jax.ShapeDtypeStruct((N, FD), F32),
            jax.ShapeDtypeStruct((N, FD), F32),
            jax.ShapeDtypeStruct((N, FD), F32),
        ],
    )(accp, accn, ys, bpb.reshape(1, H), bnb.reshape(1, H))


def _sc_deep_body(hcat, ep, en, z128, tp_out, tn_out,
                  idx2, radja, gbuf, accp, accn, gsem):
    c = lax.axis_index("c")
    s = lax.axis_index("s")
    wid = s * NC + c
    pltpu.sync_copy(z128, accp.at[pl.ds(s * ZR, ZR)])
    pltpu.sync_copy(z128, accn.at[pl.ds(s * ZR, ZR)])
    plsc.subcore_barrier()

    _seg_sum_set(ep, hcat, accp, wid, idx2, radja, gbuf, gsem)
    _seg_sum_set(en, hcat, accn, wid, idx2, radja, gbuf, gsem)

    plsc.subcore_barrier()
    pltpu.sync_copy(accp.at[pl.ds(s * WR, WR)],
                    tp_out.at[c, pl.ds(s * WR, WR)])
    pltpu.sync_copy(accn.at[pl.ds(s * WR, WR)],
                    tn_out.at[c, pl.ds(s * WR, WR)])


def _sc_deep(hcat, ep, en):
    z128 = jnp.zeros((ZR, FD), F32)
    mesh = plsc.VectorSubcoreMesh(core_axis_name="c", subcore_axis_name="s")
    fn = functools.partial(
        pl.kernel,
        mesh=mesh,
        out_type=[
            jax.ShapeDtypeStruct((NC, N, FD), F32),
            jax.ShapeDtypeStruct((NC, N, FD), F32),
        ],
        scratch_types=[
            pltpu.VMEM((2, EPW), jnp.int32),
            pltpu.VMEM((EPW,), jnp.int32),
            pltpu.VMEM((NB, CH, FD), F32),
            pltpu.VMEM_SHARED((NR, FD), F32),
            pltpu.VMEM_SHARED((NR, FD), F32),
            pltpu.SemaphoreType.DMA((NB,)),
        ],
    )(_sc_deep_body)
    return fn(hcat, ep, en, z128)


def _deep_combine_body(tp_ref, tn_ref, hcat_ref, invp_ref, invn_ref,
                       wp_ref, wn_ref, bp_ref, bn_ref, xmol_ref):
    hcat = hcat_ref[...]
    up = (tp_ref[0] + tp_ref[1] + hcat) * invp_ref[...]
    un = (tn_ref[0] + tn_ref[1] + hcat) * invn_ref[...]
    hp0 = hcat[:, :H]
    hn0 = hcat[:, H:]
    catp = jnp.concatenate([up[:, :H], un[:, H:], hp0], axis=1)
    catn = jnp.concatenate([up[:, H:], un[:, :H], hn0], axis=1)
    hp_pre = jnp.dot(catp, wp_ref[...], preferred_element_type=F32) + bp_ref[...]
    hn_pre = jnp.dot(catn, wn_ref[...], preferred_element_type=F32) + bn_ref[...]
    hp1 = jnp.tanh(_normalize_rows(hp_pre))
    hn1 = jnp.tanh(_normalize_rows(hn_pre))
    xmol_ref[...] = _normalize_rows(jnp.concatenate([hp1, hn1], axis=1))


def _deep_combine(tp, tn, hcat, invp, invn, Wpd, Wnd, bpd, bnd):
    grid = (N // BM,)
    return pl.pallas_call(
        _deep_combine_body,
        grid=grid,
        in_specs=[
            pl.BlockSpec((NC, BM, FD), lambda i: (0, i, 0)),
            pl.BlockSpec((NC, BM, FD), lambda i: (0, i, 0)),
            pl.BlockSpec((BM, FD), lambda i: (i, 0)),
            pl.BlockSpec((BM, FD), lambda i: (i, 0)),
            pl.BlockSpec((BM, FD), lambda i: (i, 0)),
            pl.BlockSpec((3 * H, H), lambda i: (0, 0)),
            pl.BlockSpec((3 * H, H), lambda i: (0, 0)),
            pl.BlockSpec((1, H), lambda i: (0, 0)),
            pl.BlockSpec((1, H), lambda i: (0, 0)),
        ],
        out_specs=pl.BlockSpec((BM, FD), lambda i: (i, 0)),
        out_shape=jax.ShapeDtypeStruct((N, FD), F32),
    )(tp, tn, hcat, invp, invn, Wpd, Wnd,
      bpd.reshape(1, H), bnd.reshape(1, H))


BP = 256
GN_ = N // BP
RPB = BP * N // 128


def _pred_body(xi_ref, xall_ref, mask_ref, lab_ref, pred_ref, loss_ref):
    i = pl.program_id(0)

    @pl.when(i == 0)
    def _init():
        loss_ref[...] = jnp.zeros((1, 1), F32)

    b = lax.dot_general(xi_ref[...], xall_ref[...],
                        (((1,), (1,)), ((), ())),
                        preferred_element_type=F32) * mask_ref[...]
    b8 = b.reshape(RPB, 128)
    pred_ref[...] = b8
    r = b8 - lab_ref[...]
    loss_ref[...] += jnp.sum(r * r).reshape(1, 1)

    @pl.when(i == GN_ - 1)
    def _fin():
        loss_ref[...] = loss_ref[...] * (1.0 / float(N * N))


def _pred_loss(xmol, label_mask, labels8):
    grid = (GN_,)
    return pl.pallas_call(
        _pred_body,
        grid=grid,
        in_specs=[
            pl.BlockSpec((BP, FD), lambda i: (i, 0)),
            pl.BlockSpec((N, FD), lambda i: (0, 0)),
            pl.BlockSpec((BP, N), lambda i: (i, 0)),
            pl.BlockSpec((RPB, 128), lambda i: (i, 0)),
        ],
        out_specs=[
            pl.BlockSpec((RPB, 128), lambda i: (i, 0)),
            pl.BlockSpec((1, 1), lambda i: (0, 0)),
        ],
        out_shape=[
            jax.ShapeDtypeStruct((N * N // 128, 128), F32),
            jax.ShapeDtypeStruct((1, 1), F32),
        ],
    )(xmol, xmol, label_mask, labels8)


def kernel(X, positive_edges, negative_edges, labels, label_mask,
           Wpb, bpb, Wnb, bnb, Wpd, bpd, Wnd, bnd):
    ep = positive_edges.astype(jnp.int32)
    en = negative_edges.astype(jnp.int32)
    Wcat = jnp.concatenate([Wpb[:D], Wnb[:D], Wpb[D:], Wnb[D:]], axis=1)

    gp, gn, ys = _project(X, Wcat)
    accp, accn = _sc_base(gp, gn, ep, en)
    hcat, invp, invn = _combine(accp, accn, ys, bpb, bnb)
    tp, tn = _sc_deep(hcat, ep, en)
    xmol = _deep_combine(tp, tn, hcat, invp, invn, Wpd, Wnd, bpd, bnd)
    pred2, lossm = _pred_loss(xmol, label_mask, labels.reshape(N * N // 128, 128))
    return (lossm[0, 0], xmol, pred2.reshape(-1))


# speedup vs baseline: 18.6102x; 1.0523x over previous
import functools

import jax
import jax.numpy as jnp
from jax import lax
from jax.experimental import pallas as pl
from jax.experimental.pallas import tpu as pltpu
from jax.experimental.pallas import tpu_sc as plsc

N = 4096
D = 2048
E = 65536
H = 64

F32 = jnp.float32

NC, NS = 2, 16
NW = NC * NS
CH = 128
EPW = E // NW
NCHUNK = EPW // CH
TRASH = N
NR = N + 128
ZR = NR // NS
WR = N // NS
FB = 128
FD = 128

BM = 512


def _normalize_rows(x):
    n = jnp.sqrt(jnp.sum(x * x, axis=1, keepdims=True))
    return x / jnp.maximum(n, 1e-12)


def _project_body(x_ref, w_ref, gp_ref, gn_ref, ys_ref):
    p = jnp.dot(x_ref[...], w_ref[...], preferred_element_type=F32)
    ones = jnp.ones((BM, FB - H), F32)
    gp_ref[...] = jnp.concatenate([p[:, :H], ones], axis=1)
    gn_ref[...] = jnp.concatenate([p[:, H:2 * H], ones], axis=1)
    ys_ref[...] = p[:, 2 * H:]


def _project(X, Wcat):
    grid = (N // BM,)
    return pl.pallas_call(
        _project_body,
        grid=grid,
        in_specs=[
            pl.BlockSpec((BM, D), lambda i: (i, 0)),
            pl.BlockSpec((D, 4 * H), lambda i: (0, 0)),
        ],
        out_specs=[
            pl.BlockSpec((BM, FB), lambda i: (i, 0)),
            pl.BlockSpec((BM, FB), lambda i: (i, 0)),
            pl.BlockSpec((BM, 2 * H), lambda i: (i, 0)),
        ],
        out_shape=[
            jax.ShapeDtypeStruct((N, FB), F32),
            jax.ShapeDtypeStruct((N, FB), F32),
            jax.ShapeDtypeStruct((N, 2 * H), F32),
        ],
    )(X, Wcat)


NB = 2
NG = NCHUNK // NB


def _seg_sum_set(eref, gref, acc, wid, idx2, radja, gbuf, gsem):
    ebase = wid * EPW
    pltpu.sync_copy(eref.at[:, pl.ds(ebase, EPW)], idx2)
    for b in range(NB):
        pltpu.async_copy(gref.at[idx2.at[1, pl.ds(b * CH, CH)]],
                         gbuf.at[b], gsem.at[b])

    def adj(j, carry):
        r = idx2[0, pl.ds(j * 16, 16)]
        cc = idx2[1, pl.ds(j * 16, 16)]
        radja[pl.ds(j * 16, 16)] = jnp.where(r == cc, TRASH, r)
        return carry

    lax.fori_loop(0, EPW // 16, adj, 0)

    def ring(g, carry):
        for b in range(NB):
            k = g * NB + b
            pltpu.make_async_copy(
                gref.at[idx2.at[1, pl.ds(k * CH, CH)]],
                gbuf.at[b], gsem.at[b]).wait()
            pltpu.sync_copy(gbuf.at[b],
                            acc.at[radja.at[pl.ds(k * CH, CH)]], add=True)

            @pl.when(k + NB < NCHUNK)
            def _issue(k=k, b=b):
                pltpu.async_copy(
                    gref.at[idx2.at[1, pl.ds((k + NB) * CH, CH)]],
                    gbuf.at[b], gsem.at[b])
        return carry

    lax.fori_loop(0, NG, ring, 0)


def _sc_base_body(gp, gn, ep, en, z80, accp_out, accn_out,
                  idx2, radja, gbuf, accp, accn, gsem):
    c = lax.axis_index("c")
    s = lax.axis_index("s")
    wid = s * NC + c
    pltpu.sync_copy(z80, accp.at[pl.ds(s * ZR, ZR)])
    pltpu.sync_copy(z80, accn.at[pl.ds(s * ZR, ZR)])
    plsc.subcore_barrier()

    _seg_sum_set(ep, gp, accp, wid, idx2, radja, gbuf, gsem)
    _seg_sum_set(en, gn, accn, wid, idx2, radja, gbuf, gsem)

    plsc.subcore_barrier()
    pltpu.sync_copy(accp.at[pl.ds(s * WR, WR)],
                    accp_out.at[c, pl.ds(s * WR, WR)])
    pltpu.sync_copy(accn.at[pl.ds(s * WR, WR)],
                    accn_out.at[c, pl.ds(s * WR, WR)])


def _sc_base(gp, gn, ep, en):
    z80 = jnp.zeros((ZR, FB), F32)
    mesh = plsc.VectorSubcoreMesh(core_axis_name="c", subcore_axis_name="s")
    fn = functools.partial(
        pl.kernel,
        mesh=mesh,
        out_type=[
            jax.ShapeDtypeStruct((NC, N, FB), F32),
            jax.ShapeDtypeStruct((NC, N, FB), F32),
        ],
        scratch_types=[
            pltpu.VMEM((2, EPW), jnp.int32),
            pltpu.VMEM((EPW,), jnp.int32),
            pltpu.VMEM((NB, CH, FB), F32),
            pltpu.VMEM_SHARED((NR, FB), F32),
            pltpu.VMEM_SHARED((NR, FB), F32),
            pltpu.SemaphoreType.DMA((NB,)),
        ],
    )(_sc_base_body)
    return fn(gp, gn, ep, en, z80)


def _combine_body(ap_ref, an_ref, ys_ref, bp_ref, bn_ref,
                  hcat_ref, invp_ref, invn_ref):
    ap = ap_ref[0] + ap_ref[1]
    an = an_ref[0] + an_ref[1]
    cp = ap[:, H:H + 1]
    cn = an[:, H:H + 1]
    hp = ap[:, :H] / jnp.maximum(cp, 1.0) + ys_ref[:, :H] + bp_ref[...]
    hn = an[:, :H] / jnp.maximum(cn, 1.0) + ys_ref[:, H:] + bn_ref[...]
    hp = jnp.tanh(_normalize_rows(hp))
    hn = jnp.tanh(_normalize_rows(hn))
    hcat_ref[...] = jnp.concatenate([hp, hn], axis=1)
    invp_ref[...] = jnp.broadcast_to(1.0 / (cp + 1.0), (BM, FD))
    invn_ref[...] = jnp.broadcast_to(1.0 / (cn + 1.0), (BM, FD))


def _combine(accp, accn, ys, bpb, bnb):
    grid = (N // BM,)
    return pl.pallas_call(
        _combine_body,
        grid=grid,
        in_specs=[
            pl.BlockSpec((NC, BM, FB), lambda i: (0, i, 0)),
            pl.BlockSpec((NC, BM, FB), lambda i: (0, i, 0)),
            pl.BlockSpec((BM, 2 * H), lambda i: (i, 0)),
            pl.BlockSpec((1, H), lambda i: (0, 0)),
            pl.BlockSpec((1, H), lambda i: (0, 0)),
        ],
        out_specs=[
            pl.BlockSpec((BM, FD), lambda i: (i, 0)),
            pl.BlockSpec((BM, FD), lambda i: (i, 0)),
            pl.BlockSpec((BM, FD), lambda i: (i, 0)),
        ],
        out_shape=[
            jax.ShapeDtypeStruct((N, FD), F32),
            jax.ShapeDtypeStruct((N, FD), F32),
            jax.ShapeDtypeStruct((N, FD), F32),
        ],
    )(accp, accn, ys, bpb.reshape(1, H), bnb.reshape(1, H))


def _sc_deep_body(hcat, ep, en, z128, tp_out, tn_out,
                  idx2, radja, gbuf, accp, accn, gsem):
    c = lax.axis_index("c")
    s = lax.axis_index("s")
    wid = s * NC + c
    pltpu.sync_copy(z128, accp.at[pl.ds(s * ZR, ZR)])
    pltpu.sync_copy(z128, accn.at[pl.ds(s * ZR, ZR)])
    plsc.subcore_barrier()

    _seg_sum_set(ep, hcat, accp, wid, idx2, radja, gbuf, gsem)
    _seg_sum_set(en, hcat, accn, wid, idx2, radja, gbuf, gsem)

    plsc.subcore_barrier()
    pltpu.sync_copy(accp.at[pl.ds(s * WR, WR)],
                    tp_out.at[c, pl.ds(s * WR, WR)])
    pltpu.sync_copy(accn.at[pl.ds(s * WR, WR)],
                    tn_out.at[c, pl.ds(s * WR, WR)])


def _sc_deep(hcat, ep, en):
    z128 = jnp.zeros((ZR, FD), F32)
    mesh = plsc.VectorSubcoreMesh(core_axis_name="c", subcore_axis_name="s")
    fn = functools.partial(
        pl.kernel,
        mesh=mesh,
        out_type=[
            jax.ShapeDtypeStruct((NC, N, FD), F32),
            jax.ShapeDtypeStruct((NC, N, FD), F32),
        ],
        scratch_types=[
            pltpu.VMEM((2, EPW), jnp.int32),
            pltpu.VMEM((EPW,), jnp.int32),
            pltpu.VMEM((NB, CH, FD), F32),
            pltpu.VMEM_SHARED((NR, FD), F32),
            pltpu.VMEM_SHARED((NR, FD), F32),
            pltpu.SemaphoreType.DMA((NB,)),
        ],
    )(_sc_deep_body)
    return fn(hcat, ep, en, z128)


def _deep_combine_body(tp_ref, tn_ref, hcat_ref, invp_ref, invn_ref,
                       wp_ref, wn_ref, bp_ref, bn_ref, xmol_ref):
    hcat = hcat_ref[...]
    up = (tp_ref[0] + tp_ref[1] + hcat) * invp_ref[...]
    un = (tn_ref[0] + tn_ref[1] + hcat) * invn_ref[...]
    hp0 = hcat[:, :H]
    hn0 = hcat[:, H:]
    catp = jnp.concatenate([up[:, :H], un[:, H:], hp0], axis=1)
    catn = jnp.concatenate([up[:, H:], un[:, :H], hn0], axis=1)
    hp_pre = jnp.dot(catp, wp_ref[...], preferred_element_type=F32) + bp_ref[...]
    hn_pre = jnp.dot(catn, wn_ref[...], preferred_element_type=F32) + bn_ref[...]
    hp1 = jnp.tanh(_normalize_rows(hp_pre))
    hn1 = jnp.tanh(_normalize_rows(hn_pre))
    xmol_ref[...] = _normalize_rows(jnp.concatenate([hp1, hn1], axis=1))


def _deep_combine(tp, tn, hcat, invp, invn, Wpd, Wnd, bpd, bnd):
    grid = (N // BM,)
    return pl.pallas_call(
        _deep_combine_body,
        grid=grid,
        in_specs=[
            pl.BlockSpec((NC, BM, FD), lambda i: (0, i, 0)),
            pl.BlockSpec((NC, BM, FD), lambda i: (0, i, 0)),
            pl.BlockSpec((BM, FD), lambda i: (i, 0)),
            pl.BlockSpec((BM, FD), lambda i: (i, 0)),
            pl.BlockSpec((BM, FD), lambda i: (i, 0)),
            pl.BlockSpec((3 * H, H), lambda i: (0, 0)),
            pl.BlockSpec((3 * H, H), lambda i: (0, 0)),
            pl.BlockSpec((1, H), lambda i: (0, 0)),
            pl.BlockSpec((1, H), lambda i: (0, 0)),
        ],
        out_specs=pl.BlockSpec((BM, FD), lambda i: (i, 0)),
        out_shape=jax.ShapeDtypeStruct((N, FD), F32),
    )(tp, tn, hcat, invp, invn, Wpd, Wnd,
      bpd.reshape(1, H), bnd.reshape(1, H))


BP = 512
GN_ = N // BP
RPB = BP * N // 128


def _pred_body(xi_ref, xall_ref, mask_ref, lab_ref, pred_ref, loss_ref):
    i = pl.program_id(0)

    @pl.when(i == 0)
    def _init():
        loss_ref[...] = jnp.zeros((1, 1), F32)

    b = lax.dot_general(xi_ref[...], xall_ref[...],
                        (((1,), (1,)), ((), ())),
                        preferred_element_type=F32) * mask_ref[...]
    b8 = b.reshape(RPB, 128)
    pred_ref[...] = b8
    r = b8 - lab_ref[...]
    loss_ref[...] += jnp.sum(r * r).reshape(1, 1)

    @pl.when(i == GN_ - 1)
    def _fin():
        loss_ref[...] = loss_ref[...] * (1.0 / float(N * N))


def _pred_loss(xmol, label_mask, labels8):
    grid = (GN_,)
    return pl.pallas_call(
        _pred_body,
        grid=grid,
        in_specs=[
            pl.BlockSpec((BP, FD), lambda i: (i, 0)),
            pl.BlockSpec((N, FD), lambda i: (0, 0)),
            pl.BlockSpec((BP, N), lambda i: (i, 0)),
            pl.BlockSpec((RPB, 128), lambda i: (i, 0)),
        ],
        out_specs=[
            pl.BlockSpec((RPB, 128), lambda i: (i, 0)),
            pl.BlockSpec((1, 1), lambda i: (0, 0)),
        ],
        out_shape=[
            jax.ShapeDtypeStruct((N * N // 128, 128), F32),
            jax.ShapeDtypeStruct((1, 1), F32),
        ],
    )(xmol, xmol, label_mask, labels8)


def kernel(X, positive_edges, negative_edges, labels, label_mask,
           Wpb, bpb, Wnb, bnb, Wpd, bpd, Wnd, bnd):
    ep = positive_edges.astype(jnp.int32)
    en = negative_edges.astype(jnp.int32)
    Wcat = jnp.concatenate([Wpb[:D], Wnb[:D], Wpb[D:], Wnb[D:]], axis=1)

    gp, gn, ys = _project(X, Wcat)
    accp, accn = _sc_base(gp, gn, ep, en)
    hcat, invp, invn = _combine(accp, accn, ys, bpb, bnb)
    tp, tn = _sc_deep(hcat, ep, en)
    xmol = _deep_combine(tp, tn, hcat, invp, invn, Wpd, Wnd, bpd, bnd)
    pred2, lossm = _pred_loss(xmol, label_mask, labels.reshape(N * N // 128, 128))
    return (lossm[0, 0], xmol, pred2.reshape(-1))
